# baseline (speedup 1.0000x reference)
import functools

import jax
import jax.numpy as jnp
from jax import lax
from jax.experimental import pallas as pl
from jax.experimental.pallas import tpu as pltpu
from jax.experimental.pallas import tpu_sc as plsc

MAX_D = 10
D_ROWS = MAX_D + 1
N_DIR = 3
N_CODES = D_ROWS * N_DIR
EMBED = 64

NW = 32
BLK = 8192
EP = 98 * BLK
PER_W = EP // NW
CHUNK = 3136
NGRP = CHUNK // 16


def _fused_table_body(ed_ref, eg_ref, dt_ref, gt_ref, w1_ref, w2_ref, b_ref,
                      out_ref):
    dist_part = jnp.dot(ed_ref[...], dt_ref[...],
                        preferred_element_type=jnp.float32)
    dir_part = jnp.dot(eg_ref[...], gt_ref[...],
                       preferred_element_type=jnp.float32)
    out_ref[...] = (
        jnp.dot(dist_part, w1_ref[...], preferred_element_type=jnp.float32)
        + jnp.dot(dir_part, w2_ref[...], preferred_element_type=jnp.float32)
        + b_ref[...]
    )


def _expand_body(codes_ref, fused_ref, out_ref):
    c = codes_ref[0]
    k = lax.broadcasted_iota(jnp.int32, (1, N_CODES), 1)
    fused = fused_ref[...]
    for r in range(8):
        col = jnp.transpose(c[r:r + 1, :])
        oh = (col == k).astype(jnp.float32)
        res = jnp.dot(oh, fused, preferred_element_type=jnp.float32)
        out_ref[pl.ds(r * 1024, 1024), :] = res


def _make_sc_codes_kernel(n_edges, n_nodes):
    assert n_edges == 800000 and EP >= n_edges
    mesh = plsc.VectorSubcoreMesh(core_axis_name="c", subcore_axis_name="s")
    TAIL_BASE = (NW - 1) * PER_W + 7 * CHUNK
    TAIL_VALID = n_edges - TAIL_BASE
    TAIL_GRPS = TAIL_VALID // 16

    @functools.partial(
        pl.kernel,
        mesh=mesh,
        compiler_params=pltpu.CompilerParams(needs_layout_passes=False,
                                             use_tc_tiling_on_sc=False),
        out_type=jax.ShapeDtypeStruct((EP,), jnp.int32),
        scratch_types=[
            pltpu.VMEM((n_nodes,), jnp.int32),
            pltpu.VMEM((CHUNK,), jnp.int32),
            pltpu.VMEM((CHUNK,), jnp.int32),
            pltpu.VMEM((CHUNK,), jnp.int32),
        ],
    )
    def sc_kernel(src_hbm, dst_hbm, dist_hbm, codes_hbm,
                  dist_v, src_v, dst_v, codes_v):
        wid = lax.axis_index("s") * 2 + lax.axis_index("c")
        pltpu.sync_copy(dist_hbm, dist_v)

        def compute_groups(g0, g1):
            def grp_body(g, carry):
                sidx = jnp.clip(src_v[pl.ds(g * 16, 16)], 0, n_nodes - 1)
                didx = jnp.clip(dst_v[pl.ds(g * 16, 16)], 0, n_nodes - 1)
                s = plsc.load_gather(dist_v, [sidx])
                t = plsc.load_gather(dist_v, [didx])
                ed = jnp.clip(jnp.minimum(s, t), 0, MAX_D)
                dirn = jnp.where(s < t, 1, jnp.where(s > t, 0, 2))
                codes_v[pl.ds(g * 16, 16)] = ed * N_DIR + dirn
                return carry
            lax.fori_loop(g0, g1, grp_body, 0, unroll=False)

        def chunk_body(i, carry):
            base = wid * PER_W + i * CHUNK
            pltpu.sync_copy(src_hbm.at[pl.ds(base, CHUNK)], src_v)
            pltpu.sync_copy(dst_hbm.at[pl.ds(base, CHUNK)], dst_v)
            compute_groups(0, NGRP)
            pltpu.sync_copy(codes_v, codes_hbm.at[pl.ds(base, CHUNK)])
            return carry

        n_full = jnp.where(wid == NW - 1, 7, 8)
        lax.fori_loop(0, n_full, chunk_body, 0, unroll=False)

        @pl.when(wid == NW - 1)
        def _tail():
            pltpu.sync_copy(src_hbm.at[pl.ds(TAIL_BASE, TAIL_VALID)],
                            src_v.at[pl.ds(0, TAIL_VALID)])
            pltpu.sync_copy(dst_hbm.at[pl.ds(TAIL_BASE, TAIL_VALID)],
                            dst_v.at[pl.ds(0, TAIL_VALID)])
            compute_groups(0, TAIL_GRPS)
            zeros = jnp.zeros((16,), jnp.int32)

            def pad_body(g, carry):
                codes_v[pl.ds(g * 16, 16)] = zeros
                return carry
            lax.fori_loop(TAIL_GRPS, NGRP, pad_body, 0, unroll=False)
            pltpu.sync_copy(codes_v, codes_hbm.at[pl.ds(TAIL_BASE, CHUNK)])

    return sc_kernel


def kernel(edge_index, distances, num_nodes, dist_table, dir_table, W, b):
    n_edges = edge_index.shape[1]
    n_nodes = distances.shape[0]

    codes = jnp.arange(N_CODES, dtype=jnp.int32)
    ed_onehot = jax.nn.one_hot(codes // N_DIR, D_ROWS, dtype=jnp.float32)
    eg_onehot = jax.nn.one_hot(codes % N_DIR, N_DIR, dtype=jnp.float32)

    fused = pl.pallas_call(
        _fused_table_body,
        out_shape=jax.ShapeDtypeStruct((N_CODES, EMBED), jnp.float32),
    )(ed_onehot, eg_onehot, dist_table, dir_table,
      W[:EMBED], W[EMBED:], b.reshape(1, EMBED))

    ei = edge_index.astype(jnp.int32)
    sc_codes = _make_sc_codes_kernel(n_edges, n_nodes)
    codes1d = sc_codes(ei[0], ei[1], distances.astype(jnp.int32))
    codes3 = codes1d.reshape(EP // BLK, 8, BLK // 8)

    out = pl.pallas_call(
        _expand_body,
        grid=(EP // BLK,),
        in_specs=[
            pl.BlockSpec((1, 8, BLK // 8), lambda i: (i, 0, 0)),
            pl.BlockSpec((N_CODES, EMBED), lambda i: (0, 0)),
        ],
        out_specs=pl.BlockSpec((BLK, EMBED), lambda i: (i, 0)),
        out_shape=jax.ShapeDtypeStruct((n_edges, EMBED), jnp.float32),
    )(codes3, fused)
    return out


# speedup vs baseline: 2.8743x; 2.8743x over previous
import functools

import jax
import jax.numpy as jnp
from jax import lax
from jax.experimental import pallas as pl
from jax.experimental.pallas import tpu as pltpu
from jax.experimental.pallas import tpu_sc as plsc

MAX_D = 10
D_ROWS = MAX_D + 1
N_DIR = 3
N_CODES = D_ROWS * N_DIR
EMBED = 64

NW = 32
BLK = 16384
EP = 49 * BLK
PER_W = EP // NW
CHUNK = 3136
NGRP = CHUNK // 16


def _fused_table_body(ed_ref, eg_ref, dt_ref, gt_ref, w1_ref, w2_ref, b_ref,
                      out_ref):
    dist_part = jnp.dot(ed_ref[...], dt_ref[...],
                        preferred_element_type=jnp.float32)
    dir_part = jnp.dot(eg_ref[...], gt_ref[...],
                       preferred_element_type=jnp.float32)
    out_ref[...] = (
        jnp.dot(dist_part, w1_ref[...], preferred_element_type=jnp.float32)
        + jnp.dot(dir_part, w2_ref[...], preferred_element_type=jnp.float32)
        + b_ref[...]
    )


def _expand_body(codes_ref, fusedt_ref, out_ref):
    c = codes_ref[0]
    ft = fusedt_ref[...]
    for r in range(8):
        cb = jnp.broadcast_to(c[r:r + 1, :], (EMBED, BLK // 8))
        got = jnp.take_along_axis(
            ft, cb, axis=1, mode=lax.GatherScatterMode.PROMISE_IN_BOUNDS)
        out_ref[:, pl.ds(r * (BLK // 8), BLK // 8)] = got


def _split_body(ei_ref, s_ref, d_ref):
    s_ref[...] = ei_ref[0, :]
    d_ref[...] = ei_ref[1, :]


def _make_sc_codes_kernel(n_edges, n_nodes):
    assert n_edges == 800000 and EP >= n_edges
    mesh = plsc.VectorSubcoreMesh(core_axis_name="c", subcore_axis_name="s")
    TAIL_BASE = (NW - 1) * PER_W + 7 * CHUNK
    TAIL_VALID = n_edges - TAIL_BASE
    TAIL_GRPS = TAIL_VALID // 16

    @functools.partial(
        pl.kernel,
        mesh=mesh,
        compiler_params=pltpu.CompilerParams(needs_layout_passes=False,
                                             use_tc_tiling_on_sc=False),
        out_type=jax.ShapeDtypeStruct((EP,), jnp.int32),
        scratch_types=[
            pltpu.VMEM((n_nodes,), jnp.int32),
            pltpu.VMEM((CHUNK,), jnp.int32),
            pltpu.VMEM((CHUNK,), jnp.int32),
            pltpu.VMEM((CHUNK,), jnp.int32),
            pltpu.SemaphoreType.DMA,
            pltpu.SemaphoreType.DMA,
        ],
    )
    def sc_kernel(src_hbm, dst_hbm, dist_hbm, codes_hbm,
                  dist_v, src_v, dst_v, codes_v, sem1, sem2):
        wid = lax.axis_index("s") * 2 + lax.axis_index("c")
        pltpu.sync_copy(dist_hbm, dist_v)

        def compute_groups(g0, g1):
            def grp_body(g, carry):
                sidx = jnp.clip(src_v[pl.ds(g * 16, 16)], 0, n_nodes - 1)
                didx = jnp.clip(dst_v[pl.ds(g * 16, 16)], 0, n_nodes - 1)
                s = plsc.load_gather(dist_v, [sidx])
                t = plsc.load_gather(dist_v, [didx])
                ed = jnp.clip(jnp.minimum(s, t), 0, MAX_D)
                dirn = jnp.where(s < t, 1, jnp.where(s > t, 0, 2))
                codes_v[pl.ds(g * 16, 16)] = ed * N_DIR + dirn
                return carry
            lax.fori_loop(g0, g1, grp_body, 0, unroll=False)

        def chunk_body(i, carry):
            base = wid * PER_W + i * CHUNK
            cs = pltpu.async_copy(src_hbm.at[pl.ds(base, CHUNK)], src_v, sem1)
            cd = pltpu.async_copy(dst_hbm.at[pl.ds(base, CHUNK)], dst_v, sem2)
            cs.wait()
            cd.wait()
            compute_groups(0, NGRP)
            pltpu.sync_copy(codes_v, codes_hbm.at[pl.ds(base, CHUNK)])
            return carry

        n_full = jnp.where(wid == NW - 1, 7, 8)
        lax.fori_loop(0, n_full, chunk_body, 0, unroll=False)

        @pl.when(wid == NW - 1)
        def _tail():
            pltpu.sync_copy(src_hbm.at[pl.ds(TAIL_BASE, TAIL_VALID)],
                            src_v.at[pl.ds(0, TAIL_VALID)])
            pltpu.sync_copy(dst_hbm.at[pl.ds(TAIL_BASE, TAIL_VALID)],
                            dst_v.at[pl.ds(0, TAIL_VALID)])
            compute_groups(0, TAIL_GRPS)
            zeros = jnp.zeros((16,), jnp.int32)

            def pad_body(g, carry):
                codes_v[pl.ds(g * 16, 16)] = zeros
                return carry
            lax.fori_loop(TAIL_GRPS, NGRP, pad_body, 0, unroll=False)
            pltpu.sync_copy(codes_v, codes_hbm.at[pl.ds(TAIL_BASE, CHUNK)])

    return sc_kernel


def kernel(edge_index, distances, num_nodes, dist_table, dir_table, W, b):
    n_edges = edge_index.shape[1]
    n_nodes = distances.shape[0]

    codes = jnp.arange(N_CODES, dtype=jnp.int32)
    ed_onehot = jax.nn.one_hot(codes // N_DIR, D_ROWS, dtype=jnp.float32)
    eg_onehot = jax.nn.one_hot(codes % N_DIR, N_DIR, dtype=jnp.float32)

    fused = pl.pallas_call(
        _fused_table_body,
        out_shape=jax.ShapeDtypeStruct((N_CODES, EMBED), jnp.float32),
    )(ed_onehot, eg_onehot, dist_table, dir_table,
      W[:EMBED], W[EMBED:], b.reshape(1, EMBED))

    ei = edge_index.astype(jnp.int32)
    sblk = 65536
    nsp = -(-n_edges // sblk)
    src, dst = pl.pallas_call(
        _split_body,
        grid=(nsp,),
        in_specs=[pl.BlockSpec((2, sblk), lambda i: (0, i))],
        out_specs=[pl.BlockSpec((sblk,), lambda i: (i,)),
                   pl.BlockSpec((sblk,), lambda i: (i,))],
        out_shape=[jax.ShapeDtypeStruct((n_edges,), jnp.int32),
                   jax.ShapeDtypeStruct((n_edges,), jnp.int32)],
    )(ei)
    sc_codes = _make_sc_codes_kernel(n_edges, n_nodes)
    codes1d = sc_codes(src, dst, distances.astype(jnp.int32))
    codes3 = codes1d.reshape(EP // BLK, 8, BLK // 8)

    out_t = pl.pallas_call(
        _expand_body,
        grid=(EP // BLK,),
        in_specs=[
            pl.BlockSpec((1, 8, BLK // 8), lambda i: (i, 0, 0)),
            pl.BlockSpec((EMBED, N_CODES), lambda i: (0, 0)),
        ],
        out_specs=pl.BlockSpec((EMBED, BLK), lambda i: (0, i)),
        out_shape=jax.ShapeDtypeStruct((EMBED, n_edges), jnp.float32),
    )(codes3, jnp.transpose(fused))
    return out_t.T
